# speedup vs baseline: 5.5402x; 5.5402x over previous
import functools
import jax
import jax.numpy as jnp
from jax import lax
from jax.experimental import pallas as pl
from jax.experimental.pallas import tpu as pltpu
from jax.experimental.pallas import tpu_sc as plsc

EMBED_DIM = 128
CHUNK = 128


@functools.cache
def _build(n_rows: int, vocab: int, d: int):
  info = plsc.get_sparse_core_info()
  nw = info.num_cores * info.num_subcores
  assert n_rows % (nw * CHUNK) == 0
  per_w = n_rows // nw
  n_chunks = per_w // CHUNK
  mesh = plsc.VectorSubcoreMesh(core_axis_name="c", subcore_axis_name="s")

  @functools.partial(
      pl.kernel,
      mesh=mesh,
      out_type=jax.ShapeDtypeStruct((n_rows, d), jnp.float32),
      scratch_types=[
          pltpu.VMEM((CHUNK,), jnp.int32),
          pltpu.VMEM((CHUNK, d), jnp.float32),
          pltpu.SemaphoreType.DMA,
      ],
  )
  def k(table_hbm, idx_hbm, out_hbm, idx_v, rows_v, sem):
    wid = lax.axis_index("s") * info.num_cores + lax.axis_index("c")
    base = wid * per_w

    def body(g, carry):
      off = base + g * CHUNK
      pltpu.sync_copy(idx_hbm.at[pl.ds(off, CHUNK)], idx_v)
      pltpu.async_copy(table_hbm.at[idx_v], rows_v, sem).wait()
      pltpu.sync_copy(rows_v, out_hbm.at[pl.ds(off, CHUNK)])
      return carry

    lax.fori_loop(0, n_chunks, body, 0)

  return k


def kernel(x, table):
  b, h = x.shape
  v, d = table.shape
  n = b * h
  x_flat = x.reshape(n).astype(jnp.int32)
  out = _build(n, v, d)(table, x_flat)
  return out.reshape(b, h, d)


# speedup vs baseline: 6.8975x; 1.2450x over previous
import functools
import jax
import jax.numpy as jnp
from jax import lax
from jax.experimental import pallas as pl
from jax.experimental.pallas import tpu as pltpu
from jax.experimental.pallas import tpu_sc as plsc

LANES = 128
GATHERS_PER_CHUNK = 2
CHUNK = LANES * GATHERS_PER_CHUNK


@functools.cache
def _build(n_rows: int, vocab: int, d: int):
  info = plsc.get_sparse_core_info()
  nw = info.num_cores * info.num_subcores
  assert n_rows % (nw * 2 * CHUNK) == 0
  per_w = n_rows // nw
  n_chunks = per_w // CHUNK
  n_pairs = n_chunks // 2
  mesh = plsc.VectorSubcoreMesh(core_axis_name="c", subcore_axis_name="s")

  @functools.partial(
      pl.kernel,
      mesh=mesh,
      out_type=jax.ShapeDtypeStruct((n_rows, d), jnp.float32),
      scratch_types=[
          pltpu.VMEM((2, CHUNK), jnp.int32),
          pltpu.VMEM((2, CHUNK, d), jnp.float32),
          pltpu.SemaphoreType.DMA,
          pltpu.SemaphoreType.DMA,
          pltpu.SemaphoreType.DMA,
      ],
  )
  def k(table_hbm, idx_hbm, out_hbm, idx_v, rows_v, gsem, wsem0, wsem1):
    wid = lax.axis_index("s") * info.num_cores + lax.axis_index("c")
    base = wid * per_w
    wsems = (wsem0, wsem1)

    def do_chunk(c, b, skip_drain):
      off = base + c * CHUNK
      pltpu.sync_copy(idx_hbm.at[pl.ds(off, CHUNK)], idx_v.at[b])
      if not skip_drain:
        pltpu.make_async_copy(rows_v.at[b],
                              out_hbm.at[pl.ds(off, CHUNK)],
                              wsems[b]).wait()
      copies = []
      for j in range(GATHERS_PER_CHUNK):
        copies.append(
            pltpu.async_copy(table_hbm.at[idx_v.at[b, pl.ds(j * LANES, LANES)]],
                             rows_v.at[b, pl.ds(j * LANES, LANES)], gsem))
      for cp in copies:
        cp.wait()
      pltpu.async_copy(rows_v.at[b], out_hbm.at[pl.ds(off, CHUNK)], wsems[b])

    do_chunk(0, 0, True)
    do_chunk(1, 1, True)

    def body(p, carry):
      do_chunk(2 * p + 0, 0, False)
      do_chunk(2 * p + 1, 1, False)
      return carry

    lax.fori_loop(1, n_pairs, body, 0)

    for b in range(2):
      c = n_chunks - 2 + b
      pltpu.make_async_copy(rows_v.at[b],
                            out_hbm.at[pl.ds(base + c * CHUNK, CHUNK)],
                            wsems[b]).wait()

  return k


def kernel(x, table):
  b, h = x.shape
  v, d = table.shape
  n = b * h
  x_flat = x.reshape(n).astype(jnp.int32)
  out = _build(n, v, d)(table, x_flat)
  return out.reshape(b, h, d)


# speedup vs baseline: 15.3291x; 2.2224x over previous
import functools
import jax
import jax.numpy as jnp
from jax import lax
from jax.experimental import pallas as pl
from jax.experimental.pallas import tpu as pltpu
from jax.experimental.pallas import tpu_sc as plsc

LANES = 128
GATHERS_PER_CHUNK = 2
CHUNK = LANES * GATHERS_PER_CHUNK


@functools.cache
def _build(n_rows: int, vocab: int, d: int):
  info = plsc.get_sparse_core_info()
  nw = info.num_cores * info.num_subcores
  assert n_rows % (nw * 2 * CHUNK) == 0
  per_w = n_rows // nw
  n_chunks = per_w // CHUNK
  n_pairs = n_chunks // 2
  mesh = plsc.VectorSubcoreMesh(core_axis_name="c", subcore_axis_name="s")

  @functools.partial(
      pl.kernel,
      mesh=mesh,
      out_type=jax.ShapeDtypeStruct((n_rows, d), jnp.float32),
      scratch_types=[
          pltpu.VMEM((2, CHUNK), jnp.int32),
          pltpu.VMEM((2, CHUNK, d), jnp.float32),
          pltpu.VMEM_SHARED((vocab, d), jnp.float32),
          pltpu.SemaphoreType.DMA,
          pltpu.SemaphoreType.DMA,
          pltpu.SemaphoreType.DMA,
      ],
  )
  def k(table_hbm, idx_hbm, out_hbm, idx_v, rows_v, table_sp, gsem,
        wsem0, wsem1):
    wid = lax.axis_index("s") * info.num_cores + lax.axis_index("c")
    base = wid * per_w
    wsems = (wsem0, wsem1)

    @pl.when(lax.axis_index("s") == 0)
    def _stage():
      pltpu.sync_copy(table_hbm, table_sp)

    plsc.subcore_barrier()

    def do_chunk(c, b, skip_drain):
      off = base + c * CHUNK
      pltpu.sync_copy(idx_hbm.at[pl.ds(off, CHUNK)], idx_v.at[b])
      if not skip_drain:
        pltpu.make_async_copy(rows_v.at[b],
                              out_hbm.at[pl.ds(off, CHUNK)],
                              wsems[b]).wait()
      copies = []
      for j in range(GATHERS_PER_CHUNK):
        copies.append(
            pltpu.async_copy(table_sp.at[idx_v.at[b, pl.ds(j * LANES, LANES)]],
                             rows_v.at[b, pl.ds(j * LANES, LANES)], gsem))
      for cp in copies:
        cp.wait()
      pltpu.async_copy(rows_v.at[b], out_hbm.at[pl.ds(off, CHUNK)], wsems[b])

    do_chunk(0, 0, True)
    do_chunk(1, 1, True)

    def body(p, carry):
      do_chunk(2 * p + 0, 0, False)
      do_chunk(2 * p + 1, 1, False)
      return carry

    lax.fori_loop(1, n_pairs, body, 0)

    for b in range(2):
      c = n_chunks - 2 + b
      pltpu.make_async_copy(rows_v.at[b],
                            out_hbm.at[pl.ds(base + c * CHUNK, CHUNK)],
                            wsems[b]).wait()

  return k


def kernel(x, table):
  b, h = x.shape
  v, d = table.shape
  n = b * h
  x_flat = x.reshape(n).astype(jnp.int32)
  out = _build(n, v, d)(table, x_flat)
  return out.reshape(b, h, d)


# speedup vs baseline: 19.4172x; 1.2667x over previous
import functools
import jax
import jax.numpy as jnp
from jax import lax
from jax.experimental import pallas as pl
from jax.experimental.pallas import tpu as pltpu
from jax.experimental.pallas import tpu_sc as plsc

CHUNK = 128
NBUF = 5


@functools.cache
def _build(n_rows: int, vocab: int, d: int):
  info = plsc.get_sparse_core_info()
  nw = info.num_cores * info.num_subcores
  per_w = n_rows // nw
  n_chunks = per_w // CHUNK
  n_groups = n_chunks // NBUF
  assert n_rows == nw * n_chunks * CHUNK and n_chunks == n_groups * NBUF
  assert n_groups >= 2
  mesh = plsc.VectorSubcoreMesh(core_axis_name="c", subcore_axis_name="s")

  @functools.partial(
      pl.kernel,
      mesh=mesh,
      out_type=jax.ShapeDtypeStruct((n_rows, d), jnp.float32),
      scratch_types=[
          pltpu.VMEM((NBUF, CHUNK), jnp.int32),
          pltpu.VMEM((NBUF, CHUNK, d), jnp.float32),
          pltpu.VMEM_SHARED((vocab, d), jnp.float32),
          pltpu.SemaphoreType.DMA((NBUF,)),
          pltpu.SemaphoreType.DMA((NBUF,)),
          pltpu.SemaphoreType.DMA((NBUF,)),
      ],
  )
  def k(table_hbm, idx_hbm, out_hbm, idx_v, rows_v, table_sp, isem, gsem,
        wsem):
    wid = lax.axis_index("s") * info.num_cores + lax.axis_index("c")
    base = wid * per_w

    @pl.when(lax.axis_index("s") == 0)
    def _stage():
      pltpu.sync_copy(table_hbm, table_sp)

    def idx_start(c, b):
      pltpu.async_copy(idx_hbm.at[pl.ds(base + c * CHUNK, CHUNK)],
                       idx_v.at[b], isem.at[b])

    def idx_wait(c, b):
      pltpu.make_async_copy(idx_hbm.at[pl.ds(base + c * CHUNK, CHUNK)],
                            idx_v.at[b], isem.at[b]).wait()

    def gather_start(b):
      pltpu.async_copy(table_sp.at[idx_v.at[b]], rows_v.at[b], gsem.at[b])

    def gather_wait(b):
      pltpu.make_async_copy(table_sp.at[idx_v.at[b]], rows_v.at[b],
                            gsem.at[b]).wait()

    def write_start(c, b):
      pltpu.async_copy(rows_v.at[b],
                       out_hbm.at[pl.ds(base + c * CHUNK, CHUNK)],
                       wsem.at[b])

    def write_wait(c, b):
      pltpu.make_async_copy(rows_v.at[b],
                            out_hbm.at[pl.ds(base + c * CHUNK, CHUNK)],
                            wsem.at[b]).wait()

    idx_start(0, 0)
    plsc.subcore_barrier()
    for b in range(NBUF):
      idx_wait(b, b)
      gather_start(b)
      if b + 1 < n_chunks:
        idx_start(b + 1, (b + 1) % NBUF)
      if b >= 1:
        gather_wait(b - 1)
        write_start(b - 1, b - 1)

    def body(g, carry):
      c0 = g * NBUF
      for b in range(NBUF):
        c = c0 + b
        write_wait(c - NBUF, b)
        idx_wait(c, b)
        gather_start(b)

        @pl.when(c + 1 < n_chunks)
        def _():
          idx_start(c + 1, (b + 1) % NBUF)

        prev = (b - 1) % NBUF
        gather_wait(prev)
        write_start(c - 1, prev)
      return carry

    lax.fori_loop(1, n_groups, body, 0)

    last = n_chunks - 1
    gather_wait(last % NBUF)
    write_start(last, last % NBUF)
    for c in range(n_chunks - NBUF, n_chunks):
      write_wait(c, c % NBUF)

  return k


def kernel(x, table):
  b, h = x.shape
  v, d = table.shape
  n = b * h
  x_flat = x.reshape(n).astype(jnp.int32)
  out = _build(n, v, d)(table, x_flat)
  return out.reshape(b, h, d)
